# Initial kernel scaffold; baseline (speedup 1.0000x reference)
import jax
import jax.numpy as jnp
from jax.experimental import pallas as pl


def kernel(pos, res_type, emb_table, Wn, bn, We, be, A, bA, B, bB, C, bC, D, bD, E, bE):
    raise NotImplementedError("write your pallas kernel here")


# speedup vs baseline: 1.8196x; 1.8196x over previous
import functools

import jax
import jax.numpy as jnp
from jax import lax
from jax.experimental import pallas as pl
from jax.experimental.pallas import tpu as pltpu
from jax.experimental.pallas import tpu_sc as plsc

HID = 128
K = 32
NRBF = 16
VOCAB = 21
RADIUS = 8.0
R2 = RADIUS * RADIUS
GAMMA = RADIUS / NRBF
BIG = 1e9
BLK = 128
EBLK = 512


def _graph_kernel(n_real, np_pad, px_ref, py_ref, pz_ref,
                  tx_ref, ty_ref, tz_ref, idx_ref, d2_ref):
    i = pl.program_id(0)
    px, py, pz = px_ref[...], py_ref[...], pz_ref[...]
    tx, ty, tz = tx_ref[...], ty_ref[...], tz_ref[...]
    x2c = tx * tx + ty * ty + tz * tz
    x2r = px * px + py * py + pz * pz
    bf = lambda v: v.astype(jnp.bfloat16).astype(jnp.float32)
    dot = bf(px) * bf(tx) + bf(py) * bf(ty) + bf(pz) * bf(tz)
    d2 = (x2r + x2c) - 2.0 * dot
    col = lax.broadcasted_iota(jnp.int32, (BLK, np_pad), 1)
    row = i * BLK + lax.broadcasted_iota(jnp.int32, (BLK, np_pad), 0)
    ok = (d2 <= R2) & (col != row) & (col < n_real)
    d2 = jnp.where(ok, d2, BIG)
    idxs, d2s = [], []
    for _ in range(K):
        m = jnp.min(d2, axis=1, keepdims=True)
        am = jnp.min(jnp.where(d2 == m, col, np_pad), axis=1, keepdims=True)
        idxs.append(am)
        d2s.append(m)
        d2 = jnp.where(col == am, BIG, d2)
    idx_ref[...] = jnp.concatenate(idxs, axis=1)
    d2_ref[...] = jnp.concatenate(d2s, axis=1)


def _init_h_kernel(rt_ref, emb_ref, wn_ref, bn_ref, h_ref):
    w2 = jnp.dot(emb_ref[...], wn_ref[...], preferred_element_type=jnp.float32)
    rt = rt_ref[...]
    oh = (rt == lax.broadcasted_iota(jnp.int32, (BLK, VOCAB), 1)).astype(jnp.float32)
    h_ref[...] = jnp.dot(oh, w2, preferred_element_type=jnp.float32, precision=lax.Precision.HIGHEST) + bn_ref[...]


def _init_e_kernel(d2_ref, we_ref, be_ref, e_ref, m_ref):
    d2 = d2_ref[...]
    m_ref[...] = (d2 <= R2).astype(jnp.float32)
    dist = jnp.sqrt(jnp.clip(d2, 0.0, BIG) + 1e-12)
    cent = lax.broadcasted_iota(jnp.int32, (EBLK, NRBF), 1).astype(
        jnp.float32) * (RADIUS / (NRBF - 1))
    phi = jnp.exp(-jnp.square(dist - cent) * (1.0 / (GAMMA * GAMMA)))
    e_ref[...] = jnp.dot(phi, we_ref[...], preferred_element_type=jnp.float32) + be_ref[...]


def _proj_kernel(h_ref, wa_ref, we_ref, wd_ref, wb_ref,
                 ba_ref, be_ref, bd_ref, bb_ref, pd_ref, pg_ref):
    h = h_ref[...]
    pd_ref[:, :HID] = jnp.dot(h, wa_ref[...], preferred_element_type=jnp.float32) + ba_ref[...]
    pd_ref[:, HID:] = jnp.dot(h, we_ref[...], preferred_element_type=jnp.float32) + be_ref[...]
    pg_ref[:, :HID] = jnp.dot(h, wd_ref[...], preferred_element_type=jnp.float32) + bd_ref[...]
    pg_ref[:, HID:] = jnp.dot(h, wb_ref[...], preferred_element_type=jnp.float32) + bb_ref[...]


def _layer_kernel(h_ref, pd_ref, e_ref, gs_ref, m_ref, wc_ref, bc_ref,
                  hn_ref, en_ref):
    e = e_ref[...]
    gs = gs_ref[...]
    ehat = jnp.dot(e, wc_ref[...], preferred_element_type=jnp.float32) + bc_ref[...]
    ehat = ehat + gs[:, :HID]
    hE = pd_ref[:, HID:]
    ehat = ehat + jnp.reshape(
        jnp.broadcast_to(hE[:, None, :], (BLK, K, HID)), (BLK * K, HID))
    sig = jax.nn.sigmoid(ehat) * m_ref[...]
    msg = sig * gs[:, HID:]
    num = jnp.sum(jnp.reshape(msg, (BLK, K, HID)), axis=1)
    den = jnp.sum(jnp.reshape(sig, (BLK, K, HID)), axis=1)
    hA = pd_ref[:, :HID]
    hn_ref[...] = h_ref[...] + jax.nn.relu(hA + num / (den + 1e-6))
    en_ref[...] = e + jax.nn.relu(ehat)


def _sc_gather(table, src, ep):
    nw = 32
    per_w = ep // nw
    ch = 64
    width = table.shape[1]
    mesh = plsc.VectorSubcoreMesh(core_axis_name="c", subcore_axis_name="s")

    @functools.partial(
        pl.kernel,
        out_type=jax.ShapeDtypeStruct((ep, width), table.dtype),
        mesh=mesh,
        scratch_types=[
            pltpu.VMEM((ch,), jnp.int32),
            pltpu.VMEM((ch, width), table.dtype),
            pltpu.SemaphoreType.DMA,
        ],
    )
    def gk(tab_hbm, src_hbm, out_hbm, idx_v, rows_v, sem):
        wid = lax.axis_index("s") * 2 + lax.axis_index("c")
        base = wid * per_w

        @pl.loop(0, per_w, step=ch)
        def _(off):
            pltpu.sync_copy(src_hbm.at[pl.ds(base + off, ch)], idx_v)
            pltpu.async_copy(tab_hbm.at[idx_v], rows_v, sem).wait()
            pltpu.sync_copy(rows_v, out_hbm.at[pl.ds(base + off, ch)])

    return gk(table, src)


def kernel(pos, res_type, emb_table, Wn, bn, We, be, A, bA, B, bB, C, bC, D, bD, E, bE):
    n = pos.shape[0]
    np_pad = ((n + BLK - 1) // BLK) * BLK
    nblk = np_pad // BLK
    ep = np_pad * K

    pos = pos.astype(jnp.float32)
    posp = jnp.concatenate(
        [pos, jnp.full((np_pad - n, 3), 1e6, jnp.float32)], axis=0)
    px, py, pz = posp[:, 0:1], posp[:, 1:2], posp[:, 2:3]
    rt = jnp.concatenate(
        [res_type.astype(jnp.int32), jnp.zeros((np_pad - n,), jnp.int32)]
    ).reshape(np_pad, 1)

    nbr, nd2 = pl.pallas_call(
        functools.partial(_graph_kernel, n, np_pad),
        grid=(nblk,),
        in_specs=[
            pl.BlockSpec((BLK, 1), lambda i: (i, 0)),
            pl.BlockSpec((BLK, 1), lambda i: (i, 0)),
            pl.BlockSpec((BLK, 1), lambda i: (i, 0)),
            pl.BlockSpec((1, np_pad), lambda i: (0, 0)),
            pl.BlockSpec((1, np_pad), lambda i: (0, 0)),
            pl.BlockSpec((1, np_pad), lambda i: (0, 0)),
        ],
        out_specs=[
            pl.BlockSpec((BLK, K), lambda i: (i, 0)),
            pl.BlockSpec((BLK, K), lambda i: (i, 0)),
        ],
        out_shape=[
            jax.ShapeDtypeStruct((np_pad, K), jnp.int32),
            jax.ShapeDtypeStruct((np_pad, K), jnp.float32),
        ],
        compiler_params=pltpu.CompilerParams(
            dimension_semantics=("parallel",)),
    )(px, py, pz, px.T, py.T, pz.T)

    h = pl.pallas_call(
        _init_h_kernel,
        grid=(nblk,),
        in_specs=[
            pl.BlockSpec((BLK, 1), lambda i: (i, 0)),
            pl.BlockSpec((VOCAB, emb_table.shape[1]), lambda i: (0, 0)),
            pl.BlockSpec((emb_table.shape[1], HID), lambda i: (0, 0)),
            pl.BlockSpec((1, HID), lambda i: (0, 0)),
        ],
        out_specs=pl.BlockSpec((BLK, HID), lambda i: (i, 0)),
        out_shape=jax.ShapeDtypeStruct((np_pad, HID), jnp.float32),
        compiler_params=pltpu.CompilerParams(
            dimension_semantics=("parallel",)),
    )(rt, emb_table.astype(jnp.float32), Wn, bn.reshape(1, HID))

    d2col = nd2.reshape(ep, 1)
    src = nbr.reshape(ep)

    e, emask = pl.pallas_call(
        _init_e_kernel,
        grid=(ep // EBLK,),
        in_specs=[
            pl.BlockSpec((EBLK, 1), lambda i: (i, 0)),
            pl.BlockSpec((NRBF, HID), lambda i: (0, 0)),
            pl.BlockSpec((1, HID), lambda i: (0, 0)),
        ],
        out_specs=[
            pl.BlockSpec((EBLK, HID), lambda i: (i, 0)),
            pl.BlockSpec((EBLK, 1), lambda i: (i, 0)),
        ],
        out_shape=[
            jax.ShapeDtypeStruct((ep, HID), jnp.float32),
            jax.ShapeDtypeStruct((ep, 1), jnp.float32),
        ],
        compiler_params=pltpu.CompilerParams(
            dimension_semantics=("parallel",)),
    )(d2col, We, be.reshape(1, HID))

    wspec = pl.BlockSpec((HID, HID), lambda i: (0, 0))
    bspec = pl.BlockSpec((1, HID), lambda i: (0, 0))
    num_layers = A.shape[0]
    for l in range(num_layers):
        pd, pg = pl.pallas_call(
            _proj_kernel,
            grid=(nblk,),
            in_specs=[pl.BlockSpec((BLK, HID), lambda i: (i, 0))]
            + [wspec] * 4 + [bspec] * 4,
            out_specs=[
                pl.BlockSpec((BLK, 2 * HID), lambda i: (i, 0)),
                pl.BlockSpec((BLK, 2 * HID), lambda i: (i, 0)),
            ],
            out_shape=[
                jax.ShapeDtypeStruct((np_pad, 2 * HID), jnp.float32),
                jax.ShapeDtypeStruct((np_pad, 2 * HID), jnp.float32),
            ],
            compiler_params=pltpu.CompilerParams(
                dimension_semantics=("parallel",)),
        )(h, A[l], E[l], D[l], B[l],
          bA[l].reshape(1, HID), bE[l].reshape(1, HID),
          bD[l].reshape(1, HID), bB[l].reshape(1, HID))

        gs = _sc_gather(pg, src, ep)

        h, e = pl.pallas_call(
            _layer_kernel,
            grid=(nblk,),
            in_specs=[
                pl.BlockSpec((BLK, HID), lambda i: (i, 0)),
                pl.BlockSpec((BLK, 2 * HID), lambda i: (i, 0)),
                pl.BlockSpec((BLK * K, HID), lambda i: (i, 0)),
                pl.BlockSpec((BLK * K, 2 * HID), lambda i: (i, 0)),
                pl.BlockSpec((BLK * K, 1), lambda i: (i, 0)),
                wspec, bspec,
            ],
            out_specs=[
                pl.BlockSpec((BLK, HID), lambda i: (i, 0)),
                pl.BlockSpec((BLK * K, HID), lambda i: (i, 0)),
            ],
            out_shape=[
                jax.ShapeDtypeStruct((np_pad, HID), jnp.float32),
                jax.ShapeDtypeStruct((ep, HID), jnp.float32),
            ],
            compiler_params=pltpu.CompilerParams(
                dimension_semantics=("parallel",)),
        )(h, pd, e, gs, emask, C[l], bC[l].reshape(1, HID))

    return h[:n]


# speedup vs baseline: 1.8247x; 1.0028x over previous
import functools

import jax
import jax.numpy as jnp
from jax import lax
from jax.experimental import pallas as pl
from jax.experimental.pallas import tpu as pltpu
from jax.experimental.pallas import tpu_sc as plsc

HID = 128
K = 32
NRBF = 16
VOCAB = 21
RADIUS = 8.0
R2 = RADIUS * RADIUS
GAMMA = RADIUS / NRBF
BIG = 1e9
BLK = 128
EBLK = 512


def _graph_kernel(n_real, np_pad, px_ref, py_ref, pz_ref,
                  tx_ref, ty_ref, tz_ref, idx_ref, d2_ref):
    i = pl.program_id(0)
    px, py, pz = px_ref[...], py_ref[...], pz_ref[...]
    tx, ty, tz = tx_ref[...], ty_ref[...], tz_ref[...]
    x2c = tx * tx + ty * ty + tz * tz
    x2r = px * px + py * py + pz * pz
    bf = lambda v: v.astype(jnp.bfloat16).astype(jnp.float32)
    dot = bf(px) * bf(tx) + bf(py) * bf(ty) + bf(pz) * bf(tz)
    d2 = (x2r + x2c) - 2.0 * dot
    col = lax.broadcasted_iota(jnp.int32, (BLK, np_pad), 1)
    row = i * BLK + lax.broadcasted_iota(jnp.int32, (BLK, np_pad), 0)
    ok = (d2 <= R2) & (col != row) & (col < n_real)
    d2 = jnp.where(ok, d2, BIG)
    idxs, d2s = [], []
    for _ in range(K):
        m = jnp.min(d2, axis=1, keepdims=True)
        am = jnp.min(jnp.where(d2 == m, col, np_pad), axis=1, keepdims=True)
        idxs.append(am)
        d2s.append(m)
        d2 = jnp.where(col == am, BIG, d2)
    idx_ref[...] = jnp.concatenate(idxs, axis=1)
    d2_ref[...] = jnp.concatenate(d2s, axis=1)


def _init_h_kernel(rt_ref, emb_ref, wn_ref, bn_ref, h_ref):
    w2 = jnp.dot(emb_ref[...], wn_ref[...], preferred_element_type=jnp.float32)
    rt = rt_ref[...]
    oh = (rt == lax.broadcasted_iota(jnp.int32, (BLK, VOCAB), 1)).astype(jnp.float32)
    h_ref[...] = jnp.dot(oh, w2, preferred_element_type=jnp.float32, precision=lax.Precision.HIGHEST) + bn_ref[...]


def _init_e_kernel(d2_ref, we_ref, be_ref, e_ref, m_ref):
    d2 = d2_ref[...]
    m_ref[...] = (d2 <= R2).astype(jnp.float32)
    dist = jnp.sqrt(jnp.clip(d2, 0.0, BIG) + 1e-12)
    cent = lax.broadcasted_iota(jnp.int32, (EBLK, NRBF), 1).astype(
        jnp.float32) * (RADIUS / (NRBF - 1))
    phi = jnp.exp(-jnp.square(dist - cent) * (1.0 / (GAMMA * GAMMA)))
    e_ref[...] = jnp.dot(phi, we_ref[...], preferred_element_type=jnp.float32) + be_ref[...]


def _proj_kernel(h_ref, wa_ref, we_ref, wd_ref, wb_ref,
                 ba_ref, be_ref, bd_ref, bb_ref, pd_ref, pg_ref):
    h = h_ref[...]
    pd_ref[:, :HID] = jnp.dot(h, wa_ref[...], preferred_element_type=jnp.float32) + ba_ref[...]
    pd_ref[:, HID:] = jnp.dot(h, we_ref[...], preferred_element_type=jnp.float32) + be_ref[...]
    pg_ref[:, :HID] = jnp.dot(h, wd_ref[...], preferred_element_type=jnp.float32) + bd_ref[...]
    pg_ref[:, HID:] = jnp.dot(h, wb_ref[...], preferred_element_type=jnp.float32) + bb_ref[...]


def _layer_kernel(h_ref, pd_ref, e_ref, gs_ref, m_ref, wc_ref, bc_ref,
                  hn_ref, en_ref):
    e = e_ref[...]
    gs = gs_ref[...].astype(jnp.float32)
    ehat = jnp.dot(e, wc_ref[...], preferred_element_type=jnp.float32) + bc_ref[...]
    ehat = ehat + gs[:, :HID]
    hE = pd_ref[:, HID:]
    ehat = ehat + jnp.reshape(
        jnp.broadcast_to(hE[:, None, :], (BLK, K, HID)), (BLK * K, HID))
    sig = jax.nn.sigmoid(ehat) * m_ref[...]
    msg = sig * gs[:, HID:]
    num = jnp.sum(jnp.reshape(msg, (BLK, K, HID)), axis=1)
    den = jnp.sum(jnp.reshape(sig, (BLK, K, HID)), axis=1)
    hA = pd_ref[:, :HID]
    hn_ref[...] = h_ref[...] + jax.nn.relu(hA + num / (den + 1e-6))
    en_ref[...] = e + jax.nn.relu(ehat)


def _sc_gather(table, src3, ep):
    nw = 32
    per_w = ep // nw
    ch = 64
    nch = per_w // ch
    nbuf = 4
    width = table.shape[1]
    mesh = plsc.VectorSubcoreMesh(core_axis_name="c", subcore_axis_name="s")

    @functools.partial(
        pl.kernel,
        out_type=jax.ShapeDtypeStruct((ep, width), table.dtype),
        mesh=mesh,
        scratch_types=[pltpu.VMEM((nch, ch), jnp.int32)]
        + [pltpu.VMEM((ch, width), table.dtype)] * nbuf
        + [pltpu.SemaphoreType.DMA] * (2 * nbuf),
    )
    def gk(tab_hbm, src_hbm, out_hbm, idx_v, *bufsem):
        bufs = bufsem[:nbuf]
        gsem = bufsem[nbuf:2 * nbuf]
        osem = bufsem[2 * nbuf:]
        wid = lax.axis_index("s") * 2 + lax.axis_index("c")
        base = wid * per_w
        pltpu.sync_copy(src_hbm.at[wid], idx_v)
        for b in range(nbuf):
            pltpu.async_copy(tab_hbm.at[idx_v.at[b]], bufs[b], gsem[b])

        @pl.loop(0, nch, step=nbuf)
        def _(j):
            for b in range(nbuf):
                c = j + b

                @pl.when(c < nch)
                def _():
                    pltpu.make_async_copy(
                        tab_hbm.at[idx_v.at[c]], bufs[b], gsem[b]).wait()
                    pltpu.async_copy(
                        bufs[b], out_hbm.at[pl.ds(base + c * ch, ch)], osem[b])

                @pl.when(c + nbuf < nch)
                def _():
                    pltpu.make_async_copy(
                        bufs[b], out_hbm.at[pl.ds(base + c * ch, ch)],
                        osem[b]).wait()
                    pltpu.async_copy(
                        tab_hbm.at[idx_v.at[c + nbuf]], bufs[b], gsem[b])

        for b in range(nbuf):
            c = nch - nbuf + b
            if c >= 0:
                pltpu.make_async_copy(
                    bufs[b], out_hbm.at[pl.ds(base + c * ch, ch)],
                    osem[b]).wait()

    return gk(table, src3)


def kernel(pos, res_type, emb_table, Wn, bn, We, be, A, bA, B, bB, C, bC, D, bD, E, bE):
    n = pos.shape[0]
    np_pad = ((n + BLK - 1) // BLK) * BLK
    nblk = np_pad // BLK
    ep = np_pad * K

    pos = pos.astype(jnp.float32)
    posp = jnp.concatenate(
        [pos, jnp.full((np_pad - n, 3), 1e6, jnp.float32)], axis=0)
    px, py, pz = posp[:, 0:1], posp[:, 1:2], posp[:, 2:3]
    rt = jnp.concatenate(
        [res_type.astype(jnp.int32), jnp.zeros((np_pad - n,), jnp.int32)]
    ).reshape(np_pad, 1)

    nbr, nd2 = pl.pallas_call(
        functools.partial(_graph_kernel, n, np_pad),
        grid=(nblk,),
        in_specs=[
            pl.BlockSpec((BLK, 1), lambda i: (i, 0)),
            pl.BlockSpec((BLK, 1), lambda i: (i, 0)),
            pl.BlockSpec((BLK, 1), lambda i: (i, 0)),
            pl.BlockSpec((1, np_pad), lambda i: (0, 0)),
            pl.BlockSpec((1, np_pad), lambda i: (0, 0)),
            pl.BlockSpec((1, np_pad), lambda i: (0, 0)),
        ],
        out_specs=[
            pl.BlockSpec((BLK, K), lambda i: (i, 0)),
            pl.BlockSpec((BLK, K), lambda i: (i, 0)),
        ],
        out_shape=[
            jax.ShapeDtypeStruct((np_pad, K), jnp.int32),
            jax.ShapeDtypeStruct((np_pad, K), jnp.float32),
        ],
        compiler_params=pltpu.CompilerParams(
            dimension_semantics=("parallel",)),
    )(px, py, pz, px.T, py.T, pz.T)

    h = pl.pallas_call(
        _init_h_kernel,
        grid=(nblk,),
        in_specs=[
            pl.BlockSpec((BLK, 1), lambda i: (i, 0)),
            pl.BlockSpec((VOCAB, emb_table.shape[1]), lambda i: (0, 0)),
            pl.BlockSpec((emb_table.shape[1], HID), lambda i: (0, 0)),
            pl.BlockSpec((1, HID), lambda i: (0, 0)),
        ],
        out_specs=pl.BlockSpec((BLK, HID), lambda i: (i, 0)),
        out_shape=jax.ShapeDtypeStruct((np_pad, HID), jnp.float32),
        compiler_params=pltpu.CompilerParams(
            dimension_semantics=("parallel",)),
    )(rt, emb_table.astype(jnp.float32), Wn, bn.reshape(1, HID))

    d2col = nd2.reshape(ep, 1)
    src3 = nbr.reshape(32, (ep // 32) // 64, 64)

    e, emask = pl.pallas_call(
        _init_e_kernel,
        grid=(ep // EBLK,),
        in_specs=[
            pl.BlockSpec((EBLK, 1), lambda i: (i, 0)),
            pl.BlockSpec((NRBF, HID), lambda i: (0, 0)),
            pl.BlockSpec((1, HID), lambda i: (0, 0)),
        ],
        out_specs=[
            pl.BlockSpec((EBLK, HID), lambda i: (i, 0)),
            pl.BlockSpec((EBLK, 1), lambda i: (i, 0)),
        ],
        out_shape=[
            jax.ShapeDtypeStruct((ep, HID), jnp.float32),
            jax.ShapeDtypeStruct((ep, 1), jnp.float32),
        ],
        compiler_params=pltpu.CompilerParams(
            dimension_semantics=("parallel",)),
    )(d2col, We, be.reshape(1, HID))

    wspec = pl.BlockSpec((HID, HID), lambda i: (0, 0))
    bspec = pl.BlockSpec((1, HID), lambda i: (0, 0))
    num_layers = A.shape[0]
    for l in range(num_layers):
        pd, pg = pl.pallas_call(
            _proj_kernel,
            grid=(nblk,),
            in_specs=[pl.BlockSpec((BLK, HID), lambda i: (i, 0))]
            + [wspec] * 4 + [bspec] * 4,
            out_specs=[
                pl.BlockSpec((BLK, 2 * HID), lambda i: (i, 0)),
                pl.BlockSpec((BLK, 2 * HID), lambda i: (i, 0)),
            ],
            out_shape=[
                jax.ShapeDtypeStruct((np_pad, 2 * HID), jnp.float32),
                jax.ShapeDtypeStruct((np_pad, 2 * HID), jnp.float32),
            ],
            compiler_params=pltpu.CompilerParams(
                dimension_semantics=("parallel",)),
        )(h, A[l], E[l], D[l], B[l],
          bA[l].reshape(1, HID), bE[l].reshape(1, HID),
          bD[l].reshape(1, HID), bB[l].reshape(1, HID))

        gs = _sc_gather(pg, src3, ep)

        h, e = pl.pallas_call(
            _layer_kernel,
            grid=(nblk,),
            in_specs=[
                pl.BlockSpec((BLK, HID), lambda i: (i, 0)),
                pl.BlockSpec((BLK, 2 * HID), lambda i: (i, 0)),
                pl.BlockSpec((BLK * K, HID), lambda i: (i, 0)),
                pl.BlockSpec((BLK * K, 2 * HID), lambda i: (i, 0)),
                pl.BlockSpec((BLK * K, 1), lambda i: (i, 0)),
                wspec, bspec,
            ],
            out_specs=[
                pl.BlockSpec((BLK, HID), lambda i: (i, 0)),
                pl.BlockSpec((BLK * K, HID), lambda i: (i, 0)),
            ],
            out_shape=[
                jax.ShapeDtypeStruct((np_pad, HID), jnp.float32),
                jax.ShapeDtypeStruct((ep, HID), jnp.float32),
            ],
            compiler_params=pltpu.CompilerParams(
                dimension_semantics=("parallel",)),
        )(h, pd, e, gs, emask, C[l], bC[l].reshape(1, HID))

    return h[:n]


# speedup vs baseline: 1.9637x; 1.0762x over previous
import functools

import jax
import jax.numpy as jnp
from jax import lax
from jax.experimental import pallas as pl
from jax.experimental.pallas import tpu as pltpu
from jax.experimental.pallas import tpu_sc as plsc

HID = 128
K = 32
NRBF = 16
VOCAB = 21
RADIUS = 8.0
R2 = RADIUS * RADIUS
GAMMA = RADIUS / NRBF
BIG = 1e9
BLK = 128
EBLK = 512


def _graph_kernel(n_real, np_pad, px_ref, py_ref, pz_ref,
                  tx_ref, ty_ref, tz_ref, idx_ref, d2_ref):
    i = pl.program_id(0)
    px, py, pz = px_ref[...], py_ref[...], pz_ref[...]
    tx, ty, tz = tx_ref[...], ty_ref[...], tz_ref[...]
    x2c = tx * tx + ty * ty + tz * tz
    x2r = px * px + py * py + pz * pz
    bf = lambda v: v.astype(jnp.bfloat16).astype(jnp.float32)
    dot = bf(px) * bf(tx) + bf(py) * bf(ty) + bf(pz) * bf(tz)
    d2 = (x2r + x2c) - 2.0 * dot
    col = lax.broadcasted_iota(jnp.int32, (BLK, np_pad), 1)
    row = i * BLK + lax.broadcasted_iota(jnp.int32, (BLK, np_pad), 0)
    ok = (d2 <= R2) & (col != row) & (col < n_real)
    d2 = jnp.where(ok, d2, BIG)
    idxs, d2s = [], []
    for _ in range(K):
        m = jnp.min(d2, axis=1, keepdims=True)
        am = jnp.min(jnp.where(d2 == m, col, np_pad), axis=1, keepdims=True)
        idxs.append(am)
        d2s.append(m)
        d2 = jnp.where(col == am, BIG, d2)
    idx_ref[...] = jnp.concatenate(idxs, axis=1)
    d2_ref[...] = jnp.concatenate(d2s, axis=1)


def _init_h_kernel(rt_ref, emb_ref, wn_ref, bn_ref, h_ref):
    w2 = jnp.dot(emb_ref[...], wn_ref[...], preferred_element_type=jnp.float32)
    rt = rt_ref[...]
    oh = (rt == lax.broadcasted_iota(jnp.int32, (BLK, VOCAB), 1)).astype(jnp.float32)
    h_ref[...] = jnp.dot(oh, w2, preferred_element_type=jnp.float32, precision=lax.Precision.HIGHEST) + bn_ref[...]


def _init_e_kernel(d2_ref, we_ref, be_ref, e_ref, m_ref):
    d2 = d2_ref[...]
    m_ref[...] = (d2 <= R2).astype(jnp.float32)
    dist = jnp.sqrt(jnp.clip(d2, 0.0, BIG) + 1e-12)
    cent = lax.broadcasted_iota(jnp.int32, (EBLK, NRBF), 1).astype(
        jnp.float32) * (RADIUS / (NRBF - 1))
    phi = jnp.exp(-jnp.square(dist - cent) * (1.0 / (GAMMA * GAMMA)))
    e_ref[...] = jnp.dot(phi, we_ref[...], preferred_element_type=jnp.float32) + be_ref[...]


def _layer_kernel(h_ref, e_ref, hs_ref, m_ref,
                  wa_ref, wb_ref, wc_ref, wd_ref, we_ref,
                  ba_ref, bb_ref, bc_ref, bd_ref, be_ref,
                  hn_ref, en_ref):
    f32 = jnp.float32
    h = h_ref[...]
    e = e_ref[...]
    hs = hs_ref[...]
    ehat = (jnp.dot(e, wc_ref[...], preferred_element_type=f32) + bc_ref[...]
            + jnp.dot(hs, wd_ref[...], preferred_element_type=f32) + bd_ref[...])
    hE = jnp.dot(h, we_ref[...], preferred_element_type=f32) + be_ref[...]
    ehat = ehat + jnp.reshape(
        jnp.broadcast_to(hE[:, None, :], (BLK, K, HID)), (BLK * K, HID))
    sig = jax.nn.sigmoid(ehat) * m_ref[...]
    msg = sig * (jnp.dot(hs, wb_ref[...], preferred_element_type=f32) + bb_ref[...])
    num = jnp.sum(jnp.reshape(msg, (BLK, K, HID)), axis=1)
    den = jnp.sum(jnp.reshape(sig, (BLK, K, HID)), axis=1)
    hA = jnp.dot(h, wa_ref[...], preferred_element_type=f32) + ba_ref[...]
    hn_ref[...] = h + jax.nn.relu(hA + num / (den + 1e-6))
    en_ref[...] = e + jax.nn.relu(ehat)


def _sc_gather(table, src3, ep):
    nw = 32
    per_w = ep // nw
    ch = 128
    nch = per_w // ch
    nbuf = 4
    width = table.shape[1]
    mesh = plsc.VectorSubcoreMesh(core_axis_name="c", subcore_axis_name="s")

    @functools.partial(
        pl.kernel,
        out_type=jax.ShapeDtypeStruct((ep, width), table.dtype),
        mesh=mesh,
        scratch_types=[pltpu.VMEM((nch, ch), jnp.int32)]
        + [pltpu.VMEM((ch, width), table.dtype)] * nbuf
        + [pltpu.SemaphoreType.DMA] * (2 * nbuf),
    )
    def gk(tab_hbm, src_hbm, out_hbm, idx_v, *bufsem):
        bufs = bufsem[:nbuf]
        gsem = bufsem[nbuf:2 * nbuf]
        osem = bufsem[2 * nbuf:]
        wid = lax.axis_index("s") * 2 + lax.axis_index("c")
        base = wid * per_w
        pltpu.sync_copy(src_hbm.at[wid], idx_v)
        for b in range(nbuf):
            pltpu.async_copy(tab_hbm.at[idx_v.at[b]], bufs[b], gsem[b])

        @pl.loop(0, nch, step=nbuf)
        def _(j):
            for b in range(nbuf):
                c = j + b

                @pl.when(c < nch)
                def _():
                    pltpu.make_async_copy(
                        tab_hbm.at[idx_v.at[c]], bufs[b], gsem[b]).wait()
                    pltpu.async_copy(
                        bufs[b], out_hbm.at[pl.ds(base + c * ch, ch)], osem[b])

                @pl.when(c + nbuf < nch)
                def _():
                    pltpu.make_async_copy(
                        bufs[b], out_hbm.at[pl.ds(base + c * ch, ch)],
                        osem[b]).wait()
                    pltpu.async_copy(
                        tab_hbm.at[idx_v.at[c + nbuf]], bufs[b], gsem[b])

        for b in range(nbuf):
            c = nch - nbuf + b
            if c >= 0:
                pltpu.make_async_copy(
                    bufs[b], out_hbm.at[pl.ds(base + c * ch, ch)],
                    osem[b]).wait()

    return gk(table, src3)


def kernel(pos, res_type, emb_table, Wn, bn, We, be, A, bA, B, bB, C, bC, D, bD, E, bE):
    n = pos.shape[0]
    np_pad = ((n + BLK - 1) // BLK) * BLK
    nblk = np_pad // BLK
    ep = np_pad * K

    pos = pos.astype(jnp.float32)
    posp = jnp.concatenate(
        [pos, jnp.full((np_pad - n, 3), 1e6, jnp.float32)], axis=0)
    px, py, pz = posp[:, 0:1], posp[:, 1:2], posp[:, 2:3]
    rt = jnp.concatenate(
        [res_type.astype(jnp.int32), jnp.zeros((np_pad - n,), jnp.int32)]
    ).reshape(np_pad, 1)

    nbr, nd2 = pl.pallas_call(
        functools.partial(_graph_kernel, n, np_pad),
        grid=(nblk,),
        in_specs=[
            pl.BlockSpec((BLK, 1), lambda i: (i, 0)),
            pl.BlockSpec((BLK, 1), lambda i: (i, 0)),
            pl.BlockSpec((BLK, 1), lambda i: (i, 0)),
            pl.BlockSpec((1, np_pad), lambda i: (0, 0)),
            pl.BlockSpec((1, np_pad), lambda i: (0, 0)),
            pl.BlockSpec((1, np_pad), lambda i: (0, 0)),
        ],
        out_specs=[
            pl.BlockSpec((BLK, K), lambda i: (i, 0)),
            pl.BlockSpec((BLK, K), lambda i: (i, 0)),
        ],
        out_shape=[
            jax.ShapeDtypeStruct((np_pad, K), jnp.int32),
            jax.ShapeDtypeStruct((np_pad, K), jnp.float32),
        ],
        compiler_params=pltpu.CompilerParams(
            dimension_semantics=("parallel",)),
    )(px, py, pz, px.T, py.T, pz.T)

    h = pl.pallas_call(
        _init_h_kernel,
        grid=(nblk,),
        in_specs=[
            pl.BlockSpec((BLK, 1), lambda i: (i, 0)),
            pl.BlockSpec((VOCAB, emb_table.shape[1]), lambda i: (0, 0)),
            pl.BlockSpec((emb_table.shape[1], HID), lambda i: (0, 0)),
            pl.BlockSpec((1, HID), lambda i: (0, 0)),
        ],
        out_specs=pl.BlockSpec((BLK, HID), lambda i: (i, 0)),
        out_shape=jax.ShapeDtypeStruct((np_pad, HID), jnp.float32),
        compiler_params=pltpu.CompilerParams(
            dimension_semantics=("parallel",)),
    )(rt, emb_table.astype(jnp.float32), Wn, bn.reshape(1, HID))

    d2col = nd2.reshape(ep, 1)
    src3 = nbr.reshape(32, (ep // 32) // 128, 128)

    e, emask = pl.pallas_call(
        _init_e_kernel,
        grid=(ep // EBLK,),
        in_specs=[
            pl.BlockSpec((EBLK, 1), lambda i: (i, 0)),
            pl.BlockSpec((NRBF, HID), lambda i: (0, 0)),
            pl.BlockSpec((1, HID), lambda i: (0, 0)),
        ],
        out_specs=[
            pl.BlockSpec((EBLK, HID), lambda i: (i, 0)),
            pl.BlockSpec((EBLK, 1), lambda i: (i, 0)),
        ],
        out_shape=[
            jax.ShapeDtypeStruct((ep, HID), jnp.float32),
            jax.ShapeDtypeStruct((ep, 1), jnp.float32),
        ],
        compiler_params=pltpu.CompilerParams(
            dimension_semantics=("parallel",)),
    )(d2col, We, be.reshape(1, HID))

    wspec = pl.BlockSpec((HID, HID), lambda i: (0, 0))
    bspec = pl.BlockSpec((1, HID), lambda i: (0, 0))
    num_layers = A.shape[0]
    for l in range(num_layers):
        hs = _sc_gather(h, src3, ep)

        h, e = pl.pallas_call(
            _layer_kernel,
            grid=(nblk,),
            in_specs=[
                pl.BlockSpec((BLK, HID), lambda i: (i, 0)),
                pl.BlockSpec((BLK * K, HID), lambda i: (i, 0)),
                pl.BlockSpec((BLK * K, HID), lambda i: (i, 0)),
                pl.BlockSpec((BLK * K, 1), lambda i: (i, 0)),
            ] + [wspec] * 5 + [bspec] * 5,
            out_specs=[
                pl.BlockSpec((BLK, HID), lambda i: (i, 0)),
                pl.BlockSpec((BLK * K, HID), lambda i: (i, 0)),
            ],
            out_shape=[
                jax.ShapeDtypeStruct((np_pad, HID), jnp.float32),
                jax.ShapeDtypeStruct((ep, HID), jnp.float32),
            ],
            compiler_params=pltpu.CompilerParams(
                dimension_semantics=("parallel",)),
        )(h, e, hs, emask,
          A[l], B[l], C[l], D[l], E[l],
          bA[l].reshape(1, HID), bB[l].reshape(1, HID), bC[l].reshape(1, HID),
          bD[l].reshape(1, HID), bE[l].reshape(1, HID))

    return h[:n]
